# Initial kernel scaffold; baseline (speedup 1.0000x reference)
import jax
import jax.numpy as jnp
from jax.experimental import pallas as pl


def kernel(num, cat, bot_W0, bot_b0, bot_W1, bot_b1, bot_W2, bot_b2, tables, top_W0, top_b0, top_W1, top_b1, top_W2, top_b2):
    raise NotImplementedError("write your pallas kernel here")


# speedup vs baseline: 6.6232x; 6.6232x over previous
import functools

import jax
import jax.numpy as jnp
from jax import lax
from jax.experimental import pallas as pl
from jax.experimental.pallas import tpu as pltpu
from jax.experimental.pallas import tpu_sc as plsc

B = 16384
NUM_DENSE = 13
F = 26
V = 100000
D = 32

_LEAK = 0.01


@functools.cache
def _make_sc_gather(n_rows: int, d: int):
    info = plsc.get_sparse_core_info()
    nw = info.num_cores * info.num_subcores
    rows_per_w = n_rows // nw
    assert rows_per_w * nw == n_rows
    chunk = 1024
    n_chunks = rows_per_w // chunk
    assert n_chunks * chunk == rows_per_w

    mesh = plsc.VectorSubcoreMesh(core_axis_name="c", subcore_axis_name="s")

    @functools.partial(
        pl.kernel,
        out_type=jax.ShapeDtypeStruct((n_rows, d), jnp.float32),
        mesh=mesh,
        scratch_types=[
            pltpu.VMEM((rows_per_w,), jnp.int32),
            pltpu.VMEM((chunk, d), jnp.float32),
            pltpu.VMEM((chunk, d), jnp.float32),
            pltpu.SemaphoreType.DMA,
            pltpu.SemaphoreType.DMA,
            pltpu.SemaphoreType.DMA,
            pltpu.SemaphoreType.DMA,
        ],
        compiler_params=pltpu.CompilerParams(use_tc_tiling_on_sc=False),
    )
    def gather_kernel(table_hbm, idx_hbm, out_hbm, idx_v, buf0, buf1,
                      gsem0, gsem1, osem0, osem1):
        wid = lax.axis_index("s") * info.num_cores + lax.axis_index("c")
        base = wid * rows_per_w
        pltpu.sync_copy(idx_hbm.at[pl.ds(base, rows_per_w)], idx_v)

        bufs = (buf0, buf1)
        gsems = (gsem0, gsem1)
        osems = (osem0, osem1)
        outs = [None, None]

        for c in range(n_chunks):
            p = c % 2
            if outs[p] is not None:
                outs[p].wait()
            g = pltpu.async_copy(
                table_hbm.at[idx_v.at[pl.ds(c * chunk, chunk)]],
                bufs[p], gsems[p])
            g.wait()
            outs[p] = pltpu.async_copy(
                bufs[p], out_hbm.at[pl.ds(base + c * chunk, chunk)], osems[p])
        for o in outs:
            if o is not None:
                o.wait()

    return gather_kernel


def _leaky(x):
    return jnp.where(x >= 0, x, _LEAK * x)


def _mlp_body(num_ref, emb_ref, bw0, bb0, bw1, bb1, bw2, bb2,
              tw0a, tw0b, tb0, tw1, tb1, tw2, tb2, out_ref):
    dot = functools.partial(jnp.dot, precision=jax.lax.Precision.HIGHEST)
    x = num_ref[...]
    x = _leaky(dot(x, bw0[...]) + bb0[...])
    x = _leaky(dot(x, bw1[...]) + bb1[...])
    x = _leaky(dot(x, bw2[...]) + bb2[...])
    e = emb_ref[...]
    h = _leaky(dot(x, tw0a[...]) + dot(e, tw0b[...]) + tb0[...])
    h = _leaky(dot(h, tw1[...]) + tb1[...])
    z = dot(h, tw2[...]) + tb2[...]
    out_ref[...] = jax.nn.sigmoid(z)


@functools.cache
def _make_mlp(bb: int):
    grid = (B // bb,)

    def row_block(i):
        return (i, 0)

    def whole(i):
        return (0, 0)

    in_specs = [
        pl.BlockSpec((bb, NUM_DENSE), row_block),
        pl.BlockSpec((bb, F * D), row_block),
        pl.BlockSpec((NUM_DENSE, 512), whole),
        pl.BlockSpec((1, 512), whole),
        pl.BlockSpec((512, 256), whole),
        pl.BlockSpec((1, 256), whole),
        pl.BlockSpec((256, 32), whole),
        pl.BlockSpec((1, 32), whole),
        pl.BlockSpec((32, 512), whole),
        pl.BlockSpec((F * D, 512), whole),
        pl.BlockSpec((1, 512), whole),
        pl.BlockSpec((512, 256), whole),
        pl.BlockSpec((1, 256), whole),
        pl.BlockSpec((256, 1), whole),
        pl.BlockSpec((1, 1), whole),
    ]
    return pl.pallas_call(
        _mlp_body,
        grid=grid,
        in_specs=in_specs,
        out_specs=pl.BlockSpec((bb, 1), row_block),
        out_shape=jax.ShapeDtypeStruct((B, 1), jnp.float32),
    )


def kernel(num, cat, bot_W0, bot_b0, bot_W1, bot_b1, bot_W2, bot_b2, tables,
           top_W0, top_b0, top_W1, top_b1, top_W2, top_b2):
    table_flat = tables.reshape(F * V, D)
    idx = (cat.astype(jnp.int32) +
           (jnp.arange(F, dtype=jnp.int32) * V)[None, :]).reshape(-1)
    emb = _make_sc_gather(B * F, D)(table_flat, idx)
    emb_flat = emb.reshape(B, F * D)

    bot = 32
    out = _make_mlp(512)(
        num, emb_flat,
        bot_W0, bot_b0.reshape(1, -1),
        bot_W1, bot_b1.reshape(1, -1),
        bot_W2, bot_b2.reshape(1, -1),
        top_W0[:bot], top_W0[bot:], top_b0.reshape(1, -1),
        top_W1, top_b1.reshape(1, -1),
        top_W2, top_b2.reshape(1, -1),
    )
    return out.reshape(B)


# speedup vs baseline: 6.6262x; 1.0005x over previous
import functools

import jax
import jax.numpy as jnp
from jax import lax
from jax.experimental import pallas as pl
from jax.experimental.pallas import tpu as pltpu
from jax.experimental.pallas import tpu_sc as plsc

B = 16384
NUM_DENSE = 13
F = 26
V = 100000
D = 32

_LEAK = 0.01


_RING = 8
_CHUNK = 416


@functools.cache
def _make_sc_gather(n_rows: int, d: int):
    info = plsc.get_sparse_core_info()
    nw = info.num_cores * info.num_subcores
    rows_per_w = n_rows // nw
    assert rows_per_w * nw == n_rows
    chunk = _CHUNK
    n_chunks = rows_per_w // chunk
    assert n_chunks * chunk == rows_per_w

    mesh = plsc.VectorSubcoreMesh(core_axis_name="c", subcore_axis_name="s")

    @functools.partial(
        pl.kernel,
        out_type=jax.ShapeDtypeStruct((n_rows, d), jnp.float32),
        mesh=mesh,
        scratch_types=[
            pltpu.VMEM((rows_per_w,), jnp.int32),
        ] + [pltpu.VMEM((chunk, d), jnp.float32) for _ in range(_RING)]
          + [pltpu.SemaphoreType.DMA for _ in range(2 * _RING)],
        compiler_params=pltpu.CompilerParams(use_tc_tiling_on_sc=False),
    )
    def gather_kernel(table_hbm, idx_hbm, out_hbm, idx_v, *rest):
        bufs = rest[:_RING]
        gsems = rest[_RING:2 * _RING]
        osems = rest[2 * _RING:3 * _RING]
        wid = lax.axis_index("s") * info.num_cores + lax.axis_index("c")
        base = wid * rows_per_w
        pltpu.sync_copy(idx_hbm.at[pl.ds(base, rows_per_w)], idx_v)

        wb = [None] * _RING
        pend = []

        def drain_one():
            c0, g0 = pend.pop(0)
            g0.wait()
            s0 = c0 % _RING
            wb[s0] = pltpu.async_copy(
                bufs[s0], out_hbm.at[pl.ds(base + c0 * chunk, chunk)],
                osems[s0])

        for c in range(n_chunks):
            s = c % _RING
            if wb[s] is not None:
                wb[s].wait()
                wb[s] = None
            pend.append((c, pltpu.async_copy(
                table_hbm.at[idx_v.at[pl.ds(c * chunk, chunk)]],
                bufs[s], gsems[s])))
            if len(pend) == _RING:
                drain_one()
        while pend:
            drain_one()
        for w in wb:
            if w is not None:
                w.wait()

    return gather_kernel


def _leaky(x):
    return jnp.where(x >= 0, x, _LEAK * x)


def _mlp_body(num_ref, emb_ref, bw0, bb0, bw1, bb1, bw2, bb2,
              tw0a, tw0b, tb0, tw1, tb1, tw2, tb2, out_ref):
    dot = functools.partial(jnp.dot, precision=jax.lax.Precision.HIGHEST)
    x = num_ref[...]
    x = _leaky(dot(x, bw0[...]) + bb0[...])
    x = _leaky(dot(x, bw1[...]) + bb1[...])
    x = _leaky(dot(x, bw2[...]) + bb2[...])
    e = emb_ref[...]
    h = _leaky(dot(x, tw0a[...]) + dot(e, tw0b[...]) + tb0[...])
    h = _leaky(dot(h, tw1[...]) + tb1[...])
    z = dot(h, tw2[...]) + tb2[...]
    out_ref[...] = jax.nn.sigmoid(z)


@functools.cache
def _make_mlp(bb: int):
    grid = (B // bb,)

    def row_block(i):
        return (i, 0)

    def whole(i):
        return (0, 0)

    in_specs = [
        pl.BlockSpec((bb, NUM_DENSE), row_block),
        pl.BlockSpec((bb, F * D), row_block),
        pl.BlockSpec((NUM_DENSE, 512), whole),
        pl.BlockSpec((1, 512), whole),
        pl.BlockSpec((512, 256), whole),
        pl.BlockSpec((1, 256), whole),
        pl.BlockSpec((256, 32), whole),
        pl.BlockSpec((1, 32), whole),
        pl.BlockSpec((32, 512), whole),
        pl.BlockSpec((F * D, 512), whole),
        pl.BlockSpec((1, 512), whole),
        pl.BlockSpec((512, 256), whole),
        pl.BlockSpec((1, 256), whole),
        pl.BlockSpec((256, 1), whole),
        pl.BlockSpec((1, 1), whole),
    ]
    return pl.pallas_call(
        _mlp_body,
        grid=grid,
        in_specs=in_specs,
        out_specs=pl.BlockSpec((bb, 1), row_block),
        out_shape=jax.ShapeDtypeStruct((B, 1), jnp.float32),
    )


def kernel(num, cat, bot_W0, bot_b0, bot_W1, bot_b1, bot_W2, bot_b2, tables,
           top_W0, top_b0, top_W1, top_b1, top_W2, top_b2):
    table_flat = tables.reshape(F * V, D)
    idx = (cat.astype(jnp.int32) +
           (jnp.arange(F, dtype=jnp.int32) * V)[None, :]).reshape(-1)
    emb = _make_sc_gather(B * F, D)(table_flat, idx)
    emb_flat = emb.reshape(B, F * D)

    bot = 32
    out = _make_mlp(512)(
        num, emb_flat,
        bot_W0, bot_b0.reshape(1, -1),
        bot_W1, bot_b1.reshape(1, -1),
        bot_W2, bot_b2.reshape(1, -1),
        top_W0[:bot], top_W0[bot:], top_b0.reshape(1, -1),
        top_W1, top_b1.reshape(1, -1),
        top_W2, top_b2.reshape(1, -1),
    )
    return out.reshape(B)


# speedup vs baseline: 13.2473x; 1.9992x over previous
import functools

import jax
import jax.numpy as jnp
from jax import lax
from jax.experimental import pallas as pl
from jax.experimental.pallas import tpu as pltpu
from jax.experimental.pallas import tpu_sc as plsc

B = 16384
NUM_DENSE = 13
F = 26
V = 100000
D = 32

_LEAK = 0.01


_V4 = V // 4


def _repack_body(t_hbm, ident_ref, out_hbm, vbuf, obuf, isem, osem):
    f = pl.program_id(0)
    ident = ident_ref[...]
    cp = pltpu.make_async_copy(t_hbm.at[f], vbuf, isem)
    cp.start()
    cp.wait()
    x = vbuf[...]
    x4 = jnp.concatenate(
        [x[:, a * _V4:(a + 1) * _V4] for a in range(4)], axis=0)
    obuf[...] = jax.lax.dot_general(
        x4, ident, (((0,), (0,)), ((), ())),
        preferred_element_type=jnp.float32)
    cpo = pltpu.make_async_copy(
        obuf, out_hbm.at[pl.ds(f * _V4, _V4)], osem)
    cpo.start()
    cpo.wait()


@functools.cache
def _make_repack():
    return pl.pallas_call(
        _repack_body,
        grid=(F,),
        in_specs=[
            pl.BlockSpec(memory_space=pl.ANY),
            pl.BlockSpec((128, 128), lambda f: (0, 0)),
        ],
        out_specs=pl.BlockSpec(memory_space=pl.ANY),
        out_shape=jax.ShapeDtypeStruct((F * V * D // 128, 128), jnp.float32),
        scratch_shapes=[
            pltpu.VMEM((D, V), jnp.float32),
            pltpu.VMEM((_V4, 128), jnp.float32),
            pltpu.SemaphoreType.DMA,
            pltpu.SemaphoreType.DMA,
        ],
        compiler_params=pltpu.CompilerParams(
            vmem_limit_bytes=100 * 1024 * 1024),
    )


_CHUNK = 1024
_RING = 2


@functools.cache
def _make_sc_gather():
    info = plsc.get_sparse_core_info()
    nw = info.num_cores * info.num_subcores
    n_rows = B * F
    rows_per_w = n_rows // nw
    n_chunks = rows_per_w // _CHUNK
    assert n_chunks * _CHUNK == rows_per_w

    mesh = plsc.VectorSubcoreMesh(core_axis_name="c", subcore_axis_name="s")

    @functools.partial(
        pl.kernel,
        out_type=jax.ShapeDtypeStruct((n_rows, D), jnp.float32),
        mesh=mesh,
        scratch_types=[
            pltpu.VMEM((rows_per_w,), jnp.int32),
        ] + [pltpu.VMEM((_CHUNK, D), jnp.float32) for _ in range(_RING)]
          + [pltpu.SemaphoreType.DMA for _ in range(2 * _RING)],
        compiler_params=pltpu.CompilerParams(use_tc_tiling_on_sc=False),
    )
    def gather_kernel(tab2d, idx_hbm, out2d, idx_v, *rest):
        bufs = rest[:_RING]
        gsems = rest[_RING:2 * _RING]
        osems = rest[2 * _RING:3 * _RING]
        wid = lax.axis_index("s") * info.num_cores + lax.axis_index("c")
        base = wid * rows_per_w
        pltpu.sync_copy(idx_hbm.at[pl.ds(base, rows_per_w)], idx_v)

        wb = [None] * _RING
        for c in range(n_chunks):
            p = c % _RING
            if wb[p] is not None:
                wb[p].wait()
            g = pltpu.async_copy(
                tab2d.at[idx_v.at[pl.ds(c * _CHUNK, _CHUNK)]],
                bufs[p], gsems[p])
            g.wait()
            wb[p] = pltpu.async_copy(
                bufs[p], out2d.at[pl.ds(base + c * _CHUNK, _CHUNK)], osems[p])
        for w in wb:
            if w is not None:
                w.wait()

    return gather_kernel


def _leaky(x):
    return jnp.where(x >= 0, x, _LEAK * x)


_DOT = functools.partial(jnp.dot, precision=jax.lax.Precision.HIGHEST)


def _bot_body(num_ref, bw0, bb0, bw1, bb1, bw2, bb2, out_ref):
    x = num_ref[...]
    x = _leaky(_DOT(x, bw0[...]) + bb0[...])
    x = _leaky(_DOT(x, bw1[...]) + bb1[...])
    out_ref[...] = _leaky(_DOT(x, bw2[...]) + bb2[...])


def _top_body(x_ref, emb_ref, tw0a, tw0b, tb0, tw1, tb1, tw2, tb2, out_ref):
    x = x_ref[...]
    e = emb_ref[...]
    h = _leaky(_DOT(x, tw0a[...]) + _DOT(e, tw0b[...]) + tb0[...])
    h = _leaky(_DOT(h, tw1[...]) + tb1[...])
    out_ref[...] = jax.nn.sigmoid(_DOT(h, tw2[...]) + tb2[...])


def _row_block(i):
    return (i, 0)


def _whole(i):
    return (0, 0)


@functools.cache
def _make_bot(bb: int):
    return pl.pallas_call(
        _bot_body,
        grid=(B // bb,),
        in_specs=[
            pl.BlockSpec((bb, NUM_DENSE), _row_block),
            pl.BlockSpec((NUM_DENSE, 512), _whole),
            pl.BlockSpec((1, 512), _whole),
            pl.BlockSpec((512, 256), _whole),
            pl.BlockSpec((1, 256), _whole),
            pl.BlockSpec((256, 32), _whole),
            pl.BlockSpec((1, 32), _whole),
        ],
        out_specs=pl.BlockSpec((bb, 32), _row_block),
        out_shape=jax.ShapeDtypeStruct((B, 32), jnp.float32),
    )


@functools.cache
def _make_top(bb: int):
    return pl.pallas_call(
        _top_body,
        grid=(B // bb,),
        in_specs=[
            pl.BlockSpec((bb, 32), _row_block),
            pl.BlockSpec((bb, F * D), _row_block),
            pl.BlockSpec((32, 512), _whole),
            pl.BlockSpec((F * D, 512), _whole),
            pl.BlockSpec((1, 512), _whole),
            pl.BlockSpec((512, 256), _whole),
            pl.BlockSpec((1, 256), _whole),
            pl.BlockSpec((256, 1), _whole),
            pl.BlockSpec((1, 1), _whole),
        ],
        out_specs=pl.BlockSpec((bb, 1), _row_block),
        out_shape=jax.ShapeDtypeStruct((B, 1), jnp.float32),
    )


def kernel(num, cat, bot_W0, bot_b0, bot_W1, bot_b1, bot_W2, bot_b2, tables,
           top_W0, top_b0, top_W1, top_b1, top_W2, top_b2):
    tab4 = _make_repack()(tables.transpose(0, 2, 1), jnp.eye(128, dtype=jnp.float32))
    c = cat.astype(jnp.int32)
    idx = (4 * (c % _V4) + c // _V4 +
           (jnp.arange(F, dtype=jnp.int32) * V)[None, :]).reshape(-1)
    emb_flat = _make_sc_gather()(tab4.reshape(F * V, D), idx).reshape(B, F * D)

    x32 = _make_bot(512)(
        num,
        bot_W0, bot_b0.reshape(1, -1),
        bot_W1, bot_b1.reshape(1, -1),
        bot_W2, bot_b2.reshape(1, -1),
    )
    bot = 32
    out = _make_top(512)(
        x32, emb_flat,
        top_W0[:bot], top_W0[bot:], top_b0.reshape(1, -1),
        top_W1, top_b1.reshape(1, -1),
        top_W2, top_b2.reshape(1, -1),
    )
    return out.reshape(B)


# speedup vs baseline: 21.1008x; 1.5928x over previous
import functools

import jax
import jax.numpy as jnp
from jax import lax
from jax.experimental import pallas as pl
from jax.experimental.pallas import tpu as pltpu
from jax.experimental.pallas import tpu_sc as plsc

B = 16384
NUM_DENSE = 13
F = 26
V = 100000
D = 32

_LEAK = 0.01


_V4 = V // 4


def _repack_body(t_ref, ident_ref, out_hbm, obuf, osem):
    f = pl.program_id(0)
    ident = ident_ref[...]
    x = t_ref[0]
    x4 = jnp.concatenate(
        [x[:, a * _V4:(a + 1) * _V4] for a in range(4)], axis=0)
    @pl.when(f > 0)
    def _():
        pltpu.make_async_copy(
            obuf, out_hbm.at[pl.ds((f - 1) * _V4, _V4)], osem).wait()
    obuf[...] = jax.lax.dot_general(
        x4, ident, (((0,), (0,)), ((), ())),
        preferred_element_type=jnp.float32)
    pltpu.make_async_copy(
        obuf, out_hbm.at[pl.ds(f * _V4, _V4)], osem).start()

    @pl.when(f == F - 1)
    def _():
        pltpu.make_async_copy(
            obuf, out_hbm.at[pl.ds(f * _V4, _V4)], osem).wait()


@functools.cache
def _make_repack():
    return pl.pallas_call(
        _repack_body,
        grid=(F,),
        in_specs=[
            pl.BlockSpec((1, D, V), lambda f: (f, 0, 0)),
            pl.BlockSpec((128, 128), lambda f: (0, 0)),
        ],
        out_specs=pl.BlockSpec(memory_space=pl.ANY),
        out_shape=jax.ShapeDtypeStruct((F * V * D // 128, 128), jnp.float32),
        scratch_shapes=[
            pltpu.VMEM((_V4, 128), jnp.float32),
            pltpu.SemaphoreType.DMA,
        ],
        compiler_params=pltpu.CompilerParams(
            vmem_limit_bytes=100 * 1024 * 1024),
    )


_CHUNK = 1024
_RING = 2


@functools.cache
def _make_sc_gather():
    info = plsc.get_sparse_core_info()
    nw = info.num_cores * info.num_subcores
    n_rows = B * F
    rows_per_w = n_rows // nw
    n_chunks = rows_per_w // _CHUNK
    assert n_chunks * _CHUNK == rows_per_w

    mesh = plsc.VectorSubcoreMesh(core_axis_name="c", subcore_axis_name="s")

    @functools.partial(
        pl.kernel,
        out_type=jax.ShapeDtypeStruct((n_rows, D), jnp.float32),
        mesh=mesh,
        scratch_types=[
            pltpu.VMEM((rows_per_w,), jnp.int32),
        ] + [pltpu.VMEM((_CHUNK, D), jnp.float32) for _ in range(_RING)]
          + [pltpu.SemaphoreType.DMA for _ in range(2 * _RING)],
        compiler_params=pltpu.CompilerParams(use_tc_tiling_on_sc=False),
    )
    def gather_kernel(tab2d, idx_hbm, out2d, idx_v, *rest):
        bufs = rest[:_RING]
        gsems = rest[_RING:2 * _RING]
        osems = rest[2 * _RING:3 * _RING]
        wid = lax.axis_index("s") * info.num_cores + lax.axis_index("c")
        base = wid * rows_per_w
        pltpu.sync_copy(idx_hbm.at[pl.ds(base, rows_per_w)], idx_v)

        wb = [None] * _RING
        for c in range(n_chunks):
            p = c % _RING
            if wb[p] is not None:
                wb[p].wait()
            g = pltpu.async_copy(
                tab2d.at[idx_v.at[pl.ds(c * _CHUNK, _CHUNK)]],
                bufs[p], gsems[p])
            g.wait()
            wb[p] = pltpu.async_copy(
                bufs[p], out2d.at[pl.ds(base + c * _CHUNK, _CHUNK)], osems[p])
        for w in wb:
            if w is not None:
                w.wait()

    return gather_kernel


def _leaky(x):
    return jnp.where(x >= 0, x, _LEAK * x)


_DOT = functools.partial(jnp.dot, precision=jax.lax.Precision.DEFAULT)


def _bot_body(num_ref, bw0, bb0, bw1, bb1, bw2, bb2, out_ref):
    x = num_ref[...]
    x = _leaky(_DOT(x, bw0[...]) + bb0[...])
    x = _leaky(_DOT(x, bw1[...]) + bb1[...])
    out_ref[...] = _leaky(_DOT(x, bw2[...]) + bb2[...])


def _top_body(x_ref, emb_ref, tw0a, tw0b, tb0, tw1, tb1, tw2, tb2, out_ref):
    x = x_ref[...]
    e = emb_ref[...]
    h = _leaky(_DOT(x, tw0a[...]) + _DOT(e, tw0b[...]) + tb0[...])
    h = _leaky(_DOT(h, tw1[...]) + tb1[...])
    out_ref[...] = jax.nn.sigmoid(_DOT(h, tw2[...]) + tb2[...])


def _row_block(i):
    return (i, 0)


def _whole(i):
    return (0, 0)


@functools.cache
def _make_bot(bb: int):
    return pl.pallas_call(
        _bot_body,
        grid=(B // bb,),
        in_specs=[
            pl.BlockSpec((bb, NUM_DENSE), _row_block),
            pl.BlockSpec((NUM_DENSE, 512), _whole),
            pl.BlockSpec((1, 512), _whole),
            pl.BlockSpec((512, 256), _whole),
            pl.BlockSpec((1, 256), _whole),
            pl.BlockSpec((256, 32), _whole),
            pl.BlockSpec((1, 32), _whole),
        ],
        out_specs=pl.BlockSpec((bb, 32), _row_block),
        out_shape=jax.ShapeDtypeStruct((B, 32), jnp.float32),
    )


@functools.cache
def _make_top(bb: int):
    return pl.pallas_call(
        _top_body,
        grid=(B // bb,),
        in_specs=[
            pl.BlockSpec((bb, 32), _row_block),
            pl.BlockSpec((bb, F * D), _row_block),
            pl.BlockSpec((32, 512), _whole),
            pl.BlockSpec((F * D, 512), _whole),
            pl.BlockSpec((1, 512), _whole),
            pl.BlockSpec((512, 256), _whole),
            pl.BlockSpec((1, 256), _whole),
            pl.BlockSpec((256, 1), _whole),
            pl.BlockSpec((1, 1), _whole),
        ],
        out_specs=pl.BlockSpec((bb, 1), _row_block),
        out_shape=jax.ShapeDtypeStruct((B, 1), jnp.float32),
    )


def kernel(num, cat, bot_W0, bot_b0, bot_W1, bot_b1, bot_W2, bot_b2, tables,
           top_W0, top_b0, top_W1, top_b1, top_W2, top_b2):
    tab4 = _make_repack()(tables.transpose(0, 2, 1), jnp.eye(128, dtype=jnp.float32))
    c = cat.astype(jnp.int32)
    idx = (4 * (c % _V4) + c // _V4 +
           (jnp.arange(F, dtype=jnp.int32) * V)[None, :]).reshape(-1)
    emb_flat = _make_sc_gather()(tab4.reshape(F * V, D), idx).reshape(B, F * D)

    x32 = _make_bot(512)(
        num,
        bot_W0, bot_b0.reshape(1, -1),
        bot_W1, bot_b1.reshape(1, -1),
        bot_W2, bot_b2.reshape(1, -1),
    )
    bot = 32
    out = _make_top(512)(
        x32, emb_flat,
        top_W0[:bot], top_W0[bot:], top_b0.reshape(1, -1),
        top_W1, top_b1.reshape(1, -1),
        top_W2, top_b2.reshape(1, -1),
    )
    return out.reshape(B)
